# baseline (speedup 1.0000x reference)
import functools
import jax
import jax.numpy as jnp
from jax import lax
from jax.experimental import pallas as pl
from jax.experimental.pallas import tpu as pltpu
from jax.experimental.pallas import tpu_sc as plsc

E = 8
TOP_K = 2
H = 1024
I = 4096
R = 81
BLK = 256
IB = 1024
NI = I // IB
NB = 40
NPAD = NB * BLK
NBP = 48
TB = 512
NEG = -3.0e38

NC = 2
NS = 16
NW = NC * NS
LANES = 16


def _route_body(x_ref, gw_ref, lg_ref, e0_ref, e1_ref, r0_ref, r1_ref,
                w0x_ref, w1x_ref, seg_ref, be_ref, xbf_ref, acc_ref):
    t = pl.program_id(0)
    f32 = jnp.float32
    lg = lax.dot_general(x_ref[...], gw_ref[...], (((1,), (1,)), ((), ())),
                         preferred_element_type=f32)
    lg_ref[...] = lg
    xb = x_ref[...].astype(jnp.bfloat16).astype(jnp.float32)
    bits = lax.bitcast_convert_type(xb, jnp.uint32)
    word = (bits[:, :H // 2] >> 16) | (bits[:, H // 2:]
                                       & jnp.uint32(0xFFFF0000))
    xbf_ref[...] = lax.bitcast_convert_type(word, jnp.int32)
    ioe = lax.broadcasted_iota(jnp.int32, (TB, E), 1)
    l0 = jnp.max(lg, axis=1, keepdims=True)
    e0 = jnp.min(jnp.where(lg == l0, ioe, E), axis=1)
    oh0 = ioe == e0[:, None]
    lg2 = jnp.where(oh0, NEG, lg)
    l1 = jnp.max(lg2, axis=1, keepdims=True)
    e1 = jnp.min(jnp.where(lg2 == l1, ioe, E), axis=1)
    oh1 = ioe == e1[:, None]
    w0 = 1.0 / (1.0 + jnp.exp(l1 - l0))
    w0x_ref[...] = jnp.broadcast_to(w0, (TB, 16))
    w1x_ref[...] = jnp.broadcast_to(1.0 - w0, (TB, 16))

    oh = oh0.astype(f32) + oh1.astype(f32)

    @pl.when(t == 0)
    def _():
        acc_ref[...] = jnp.zeros_like(acc_ref)

    ri = lax.broadcasted_iota(jnp.int32, (TB, TB), 0)
    ci = lax.broadcasted_iota(jnp.int32, (TB, TB), 1)
    ls = (ri > ci).astype(f32)
    ex = lax.dot_general(ls, oh, (((1,), (0,)), ((), ())),
                         preferred_element_type=f32) + acc_ref[...]
    rank0 = jnp.sum(jnp.where(oh0, ex, 0.0), axis=1)
    rank1 = jnp.sum(jnp.where(oh1, ex, 0.0), axis=1)
    e0_ref[...] = e0.reshape(1, 1, TB)
    e1_ref[...] = e1.reshape(1, 1, TB)
    r0_ref[...] = rank0.astype(jnp.int32).reshape(1, 1, TB)
    r1_ref[...] = rank1.astype(jnp.int32).reshape(1, 1, TB)

    counts = acc_ref[...] + jnp.sum(oh, axis=0, keepdims=True)
    acc_ref[...] = counts

    padded = jnp.floor((counts + (BLK - 1)) * (1.0 / BLK)) * BLK
    ut = (lax.broadcasted_iota(jnp.int32, (E, E), 0)
          <= lax.broadcasted_iota(jnp.int32, (E, E), 1)).astype(f32)
    bounds = lax.dot_general(padded, ut, (((1,), (0,)), ((), ())),
                             preferred_element_type=f32)
    seg = bounds - padded
    rowi = lax.broadcasted_iota(jnp.int32, (E, 16), 0)
    s16 = jnp.zeros((E, 16), f32)
    for e in range(E):
        s16 = jnp.where(rowi == e, seg[0, e], s16)
    seg_ref[...] = s16.astype(jnp.int32)

    bstart = (lax.broadcasted_iota(jnp.int32, (1, NBP), 1) * BLK).astype(f32)
    be = jnp.zeros((1, NBP), jnp.int32)
    for e in range(E):
        be = be + (bstart >= bounds[0, e]).astype(jnp.int32)
    be_ref[...] = jnp.minimum(be, E - 1).reshape(1, 1, NBP)


def _route(x, gate_w):
    T = x.shape[0]
    ntb = T // TB
    outs = pl.pallas_call(
        _route_body,
        grid=(ntb,),
        in_specs=[
            pl.BlockSpec((TB, H), lambda t: (t, 0)),
            pl.BlockSpec((E, H), lambda t: (0, 0)),
        ],
        out_specs=[
            pl.BlockSpec((TB, E), lambda t: (t, 0)),
            pl.BlockSpec((1, 1, TB), lambda t: (t, 0, 0)),
            pl.BlockSpec((1, 1, TB), lambda t: (t, 0, 0)),
            pl.BlockSpec((1, 1, TB), lambda t: (t, 0, 0)),
            pl.BlockSpec((1, 1, TB), lambda t: (t, 0, 0)),
            pl.BlockSpec((TB, 16), lambda t: (t, 0)),
            pl.BlockSpec((TB, 16), lambda t: (t, 0)),
            pl.BlockSpec((E, 16), lambda t: (0, 0)),
            pl.BlockSpec((1, 1, NBP), lambda t: (0, 0, 0)),
            pl.BlockSpec((TB, H // 2), lambda t: (t, 0)),
        ],
        out_shape=[
            jax.ShapeDtypeStruct((T, E), jnp.float32),
            jax.ShapeDtypeStruct((ntb, 1, TB), jnp.int32),
            jax.ShapeDtypeStruct((ntb, 1, TB), jnp.int32),
            jax.ShapeDtypeStruct((ntb, 1, TB), jnp.int32),
            jax.ShapeDtypeStruct((ntb, 1, TB), jnp.int32),
            jax.ShapeDtypeStruct((T, 16), jnp.float32),
            jax.ShapeDtypeStruct((T, 16), jnp.float32),
            jax.ShapeDtypeStruct((E, 16), jnp.int32),
            jax.ShapeDtypeStruct((1, 1, NBP), jnp.int32),
            jax.ShapeDtypeStruct((T, H // 2), jnp.int32),
        ],
        scratch_shapes=[pltpu.VMEM((1, E), jnp.float32)],
        compiler_params=pltpu.CompilerParams(
            dimension_semantics=("arbitrary",)),
    )(x, gate_w)
    lg, e0, e1, r0, r1, w0x, w1x, seg, be, xbf = outs
    return (lg, e0.reshape(T), e1.reshape(T), r0.reshape(T), r1.reshape(T),
            w0x, w1x, seg, be.reshape(NBP), xbf)


def _sc_build(e0, e1, r0, r1, seg, x):
    T = e0.shape[0]
    tpw = T // NW
    nch = tpw // LANES
    mesh = plsc.VectorSubcoreMesh(core_axis_name="c", subcore_axis_name="s",
                                  num_cores=NC, num_subcores=NS)

    @functools.partial(
        pl.kernel,
        out_type=[
            jax.ShapeDtypeStruct((NPAD, H // 2), jnp.int32),
            jax.ShapeDtypeStruct((T,), jnp.int32),
            jax.ShapeDtypeStruct((T,), jnp.int32),
        ],
        mesh=mesh,
        scratch_types=[
            pltpu.VMEM((tpw,), jnp.int32),
            pltpu.VMEM((tpw,), jnp.int32),
            pltpu.VMEM((tpw,), jnp.int32),
            pltpu.VMEM((tpw,), jnp.int32),
            pltpu.VMEM((tpw,), jnp.int32),
            pltpu.VMEM((tpw,), jnp.int32),
            pltpu.VMEM((E, 16), jnp.int32),
            pltpu.VMEM((LANES, H // 2), jnp.int32),
            pltpu.SemaphoreType.DMA,
        ],
    )
    def build(e0_hbm, e1_hbm, r0_hbm, r1_hbm, seg_hbm, x_hbm,
              xs_hbm, p0_hbm, p1_hbm,
              e0_v, e1_v, r0_v, r1_v, p0_v, p1_v, seg_v, xr_v, sem):
        wid = lax.axis_index("s") * NC + lax.axis_index("c")
        base = wid * tpw
        sl = pl.ds(base, tpw)
        pltpu.sync_copy(seg_hbm, seg_v)
        pltpu.sync_copy(e0_hbm.at[sl], e0_v)
        pltpu.sync_copy(e1_hbm.at[sl], e1_v)
        pltpu.sync_copy(r0_hbm.at[sl], r0_v)
        pltpu.sync_copy(r1_hbm.at[sl], r1_v)
        for c in range(nch):
            cs = pl.ds(c * LANES, LANES)
            e0c = e0_v[cs]
            e1c = e1_v[cs]
            r0c = r0_v[cs]
            r1c = r1_v[cs]
            p0c = jnp.zeros((16,), jnp.int32)
            p1c = jnp.zeros((16,), jnp.int32)
            for e in range(E):
                srow = seg_v[e, :]
                p0c = jnp.where(e0c == e, srow + r0c, p0c)
                p1c = jnp.where(e1c == e, srow + r1c, p1c)
            p0_v[cs] = p0c
            p1_v[cs] = p1c
            pltpu.sync_copy(x_hbm.at[pl.ds(base + c * LANES, LANES)], xr_v)
            pltpu.async_copy(xr_v, xs_hbm.at[p0c], sem).wait()
            pltpu.async_copy(xr_v, xs_hbm.at[p1c], sem).wait()
        pltpu.sync_copy(p0_v, p0_hbm.at[sl])
        pltpu.sync_copy(p1_v, p1_hbm.at[sl])

    return build(e0, e1, r0, r1, seg, x)


def _dgT(a, b):
    return jax.lax.dot_general(
        a, b, (((1,), (1,)), ((), ())), preferred_element_type=jnp.float32)


def _moe_body(be_ref, xs_ref, w1_ref, w3_ref, w2_ref,
              u1_ref, v1_ref, u3_ref, v3_ref, u2_ref, v2_ref, ys_ref,
              acc_ref):
    i = pl.program_id(0)
    b = pl.program_id(1)
    bf = jnp.bfloat16
    u = lax.bitcast_convert_type(xs_ref[...], jnp.uint32)
    xlo = lax.bitcast_convert_type(u << 16, jnp.float32)
    xhi = lax.bitcast_convert_type(u & jnp.uint32(0xFFFF0000), jnp.float32)
    x = jnp.concatenate([xlo, xhi], axis=1).astype(bf)
    gate = _dgT(x, w1_ref[0]) + _dgT(_dgT(x, v1_ref[0]).astype(bf), u1_ref[0])
    up = _dgT(x, w3_ref[0]) + _dgT(_dgT(x, v3_ref[0]).astype(bf), u3_ref[0])
    h = (gate * jax.nn.sigmoid(gate) * up).astype(bf)
    part = _dgT(h, w2_ref[0]) + _dgT(_dgT(h, v2_ref[0]).astype(bf), u2_ref[0])

    sl = pl.ds(b * BLK, BLK)

    @pl.when(i == 0)
    def _():
        acc_ref[sl, :] = part.astype(jnp.bfloat16)

    @pl.when(i > 0)
    def _():
        acc_ref[sl, :] += part.astype(jnp.bfloat16)

    ys_ref[...] = acc_ref[sl, :].astype(jnp.float32)


def _grouped_mlp(block_expert, xs, w1, w2, w3, u1, v1, u2, v2, u3, v3):
    grid_spec = pltpu.PrefetchScalarGridSpec(
        num_scalar_prefetch=1,
        grid=(NI, NB),
        in_specs=[
            pl.BlockSpec((BLK, H // 2), lambda i, b, be: (b, 0)),
            pl.BlockSpec((1, IB, H), lambda i, b, be: (be[b], i, 0)),
            pl.BlockSpec((1, IB, H), lambda i, b, be: (be[b], i, 0)),
            pl.BlockSpec((1, H, IB), lambda i, b, be: (be[b], 0, i)),
            pl.BlockSpec((1, IB, R), lambda i, b, be: (be[b], i, 0)),
            pl.BlockSpec((1, R, H), lambda i, b, be: (be[b], 0, 0)),
            pl.BlockSpec((1, IB, R), lambda i, b, be: (be[b], i, 0)),
            pl.BlockSpec((1, R, H), lambda i, b, be: (be[b], 0, 0)),
            pl.BlockSpec((1, H, R), lambda i, b, be: (be[b], 0, 0)),
            pl.BlockSpec((1, R, IB), lambda i, b, be: (be[b], 0, i)),
        ],
        out_specs=pl.BlockSpec((BLK, H), lambda i, b, be: (b, 0)),
        scratch_shapes=[pltpu.VMEM((NPAD, H), jnp.bfloat16)],
    )
    return pl.pallas_call(
        _moe_body,
        grid_spec=grid_spec,
        out_shape=jax.ShapeDtypeStruct((NPAD, H), jnp.float32),
        compiler_params=pltpu.CompilerParams(
            dimension_semantics=("arbitrary", "arbitrary")),
    )(block_expert, xs, w1, w3, w2, u1, v1, u3, v3, u2, v2)


def _sc_combine(ys, p0, p1, w0x, w1x):
    T = p0.shape[0]
    tpw = T // NW
    nch = tpw // LANES
    mesh = plsc.VectorSubcoreMesh(core_axis_name="c", subcore_axis_name="s",
                                  num_cores=NC, num_subcores=NS)

    @functools.partial(
        pl.kernel,
        out_type=jax.ShapeDtypeStruct((T, H), jnp.float32),
        mesh=mesh,
        scratch_types=[
            pltpu.VMEM((tpw,), jnp.int32),
            pltpu.VMEM((tpw,), jnp.int32),
            pltpu.VMEM((LANES, 16), jnp.float32),
            pltpu.VMEM((LANES, 16), jnp.float32),
            pltpu.VMEM((LANES, H), jnp.float32),
            pltpu.VMEM((LANES, H), jnp.float32),
            pltpu.VMEM((LANES, H), jnp.float32),
            pltpu.SemaphoreType.DMA,
            pltpu.SemaphoreType.DMA,
        ],
    )
    def combine(ys_hbm, p0_hbm, p1_hbm, w0x_hbm, w1x_hbm, out_hbm,
                p0_v, p1_v, wx0_v, wx1_v, g0_v, g1_v, o_v, sem0, sem1):
        wid = lax.axis_index("s") * NC + lax.axis_index("c")
        base = wid * tpw
        sl = pl.ds(base, tpw)
        pltpu.sync_copy(p0_hbm.at[sl], p0_v)
        pltpu.sync_copy(p1_hbm.at[sl], p1_v)
        for c in range(nch):
            cs = pl.ds(c * LANES, LANES)
            tok = pl.ds(base + c * LANES, LANES)
            idx0 = p0_v[cs]
            idx1 = p1_v[cs]
            cp0 = pltpu.async_copy(ys_hbm.at[idx0], g0_v, sem0)
            cp1 = pltpu.async_copy(ys_hbm.at[idx1], g1_v, sem1)
            pltpu.sync_copy(w0x_hbm.at[tok], wx0_v)
            pltpu.sync_copy(w1x_hbm.at[tok], wx1_v)
            cp0.wait()
            cp1.wait()
            for r in range(LANES):
                w0s = wx0_v[r, :]
                w1s = wx1_v[r, :]

                def body(j, _):
                    js = pl.ds(j * LANES, LANES)
                    o_v[r, js] = g0_v[r, js] * w0s + g1_v[r, js] * w1s
                    return 0

                lax.fori_loop(0, H // LANES, body, 0)
            pltpu.sync_copy(o_v, out_hbm.at[tok])

    return combine(ys, p0, p1, w0x, w1x)


def kernel(hidden_states, gate_w, w1, w2, w3, u1, v1, u2, v2, u3, v3):
    b, s, hd = hidden_states.shape
    x = hidden_states.reshape(-1, hd)

    logits, e0, e1, r0, r1, w0x, w1x, seg, be_tab, xbf = _route(x, gate_w)
    xs, p0, p1 = _sc_build(e0, e1, r0, r1, seg, xbf)
    bf = jnp.bfloat16
    ys = _grouped_mlp(be_tab[:NB], xs,
                      w1.astype(bf), w2.astype(bf), w3.astype(bf),
                      u1.astype(bf), v1.astype(bf), u2.astype(bf),
                      v2.astype(bf), u3.astype(bf), v3.astype(bf))
    final = _sc_combine(ys, p0, p1, w0x, w1x)
    return final.reshape(b, s, hd), logits


# speedup vs baseline: 1.1853x; 1.1853x over previous
import functools
import jax
import jax.numpy as jnp
from jax import lax
from jax.experimental import pallas as pl
from jax.experimental.pallas import tpu as pltpu
from jax.experimental.pallas import tpu_sc as plsc

E = 8
TOP_K = 2
H = 1024
I = 4096
R = 81
BLK = 256
IB = 1024
NI = I // IB
NB = 40
NPAD = NB * BLK
NBP = 48
TB = 512
NEG = -3.0e38

NC = 2
NS = 16
NW = NC * NS
LANES = 16


def _route_body(x_ref, gw_ref, lg_ref, e0_ref, e1_ref, r0_ref, r1_ref,
                w0x_ref, w1x_ref, seg_ref, be_ref, acc_ref):
    t = pl.program_id(0)
    f32 = jnp.float32
    lg = lax.dot_general(x_ref[...], gw_ref[...], (((1,), (1,)), ((), ())),
                         preferred_element_type=f32)
    lg_ref[...] = lg
    ioe = lax.broadcasted_iota(jnp.int32, (TB, E), 1)
    l0 = jnp.max(lg, axis=1, keepdims=True)
    e0 = jnp.min(jnp.where(lg == l0, ioe, E), axis=1)
    oh0 = ioe == e0[:, None]
    lg2 = jnp.where(oh0, NEG, lg)
    l1 = jnp.max(lg2, axis=1, keepdims=True)
    e1 = jnp.min(jnp.where(lg2 == l1, ioe, E), axis=1)
    oh1 = ioe == e1[:, None]
    w0 = 1.0 / (1.0 + jnp.exp(l1 - l0))
    w0x_ref[...] = jnp.broadcast_to(w0, (TB, 16))
    w1x_ref[...] = jnp.broadcast_to(1.0 - w0, (TB, 16))

    oh = oh0.astype(f32) + oh1.astype(f32)

    @pl.when(t == 0)
    def _():
        acc_ref[...] = jnp.zeros_like(acc_ref)

    ri = lax.broadcasted_iota(jnp.int32, (TB, TB), 0)
    ci = lax.broadcasted_iota(jnp.int32, (TB, TB), 1)
    ls = (ri > ci).astype(f32)
    ex = lax.dot_general(ls, oh, (((1,), (0,)), ((), ())),
                         preferred_element_type=f32) + acc_ref[...]
    rank0 = jnp.sum(jnp.where(oh0, ex, 0.0), axis=1)
    rank1 = jnp.sum(jnp.where(oh1, ex, 0.0), axis=1)
    e0_ref[...] = e0.reshape(1, 1, TB)
    e1_ref[...] = e1.reshape(1, 1, TB)
    r0_ref[...] = rank0.astype(jnp.int32).reshape(1, 1, TB)
    r1_ref[...] = rank1.astype(jnp.int32).reshape(1, 1, TB)

    counts = acc_ref[...] + jnp.sum(oh, axis=0, keepdims=True)
    acc_ref[...] = counts

    padded = jnp.floor((counts + (BLK - 1)) * (1.0 / BLK)) * BLK
    ut = (lax.broadcasted_iota(jnp.int32, (E, E), 0)
          <= lax.broadcasted_iota(jnp.int32, (E, E), 1)).astype(f32)
    bounds = lax.dot_general(padded, ut, (((1,), (0,)), ((), ())),
                             preferred_element_type=f32)
    seg = bounds - padded
    rowi = lax.broadcasted_iota(jnp.int32, (E, 16), 0)
    s16 = jnp.zeros((E, 16), f32)
    for e in range(E):
        s16 = jnp.where(rowi == e, seg[0, e], s16)
    seg_ref[...] = s16.astype(jnp.int32)

    bstart = (lax.broadcasted_iota(jnp.int32, (1, NBP), 1) * BLK).astype(f32)
    be = jnp.zeros((1, NBP), jnp.int32)
    for e in range(E):
        be = be + (bstart >= bounds[0, e]).astype(jnp.int32)
    be_ref[...] = jnp.minimum(be, E - 1).reshape(1, 1, NBP)


def _route(x, gate_w):
    T = x.shape[0]
    ntb = T // TB
    outs = pl.pallas_call(
        _route_body,
        grid=(ntb,),
        in_specs=[
            pl.BlockSpec((TB, H), lambda t: (t, 0)),
            pl.BlockSpec((E, H), lambda t: (0, 0)),
        ],
        out_specs=[
            pl.BlockSpec((TB, E), lambda t: (t, 0)),
            pl.BlockSpec((1, 1, TB), lambda t: (t, 0, 0)),
            pl.BlockSpec((1, 1, TB), lambda t: (t, 0, 0)),
            pl.BlockSpec((1, 1, TB), lambda t: (t, 0, 0)),
            pl.BlockSpec((1, 1, TB), lambda t: (t, 0, 0)),
            pl.BlockSpec((TB, 16), lambda t: (t, 0)),
            pl.BlockSpec((TB, 16), lambda t: (t, 0)),
            pl.BlockSpec((E, 16), lambda t: (0, 0)),
            pl.BlockSpec((1, 1, NBP), lambda t: (0, 0, 0)),
        ],
        out_shape=[
            jax.ShapeDtypeStruct((T, E), jnp.float32),
            jax.ShapeDtypeStruct((ntb, 1, TB), jnp.int32),
            jax.ShapeDtypeStruct((ntb, 1, TB), jnp.int32),
            jax.ShapeDtypeStruct((ntb, 1, TB), jnp.int32),
            jax.ShapeDtypeStruct((ntb, 1, TB), jnp.int32),
            jax.ShapeDtypeStruct((T, 16), jnp.float32),
            jax.ShapeDtypeStruct((T, 16), jnp.float32),
            jax.ShapeDtypeStruct((E, 16), jnp.int32),
            jax.ShapeDtypeStruct((1, 1, NBP), jnp.int32),
        ],
        scratch_shapes=[pltpu.VMEM((1, E), jnp.float32)],
        compiler_params=pltpu.CompilerParams(
            dimension_semantics=("arbitrary",)),
    )(x, gate_w)
    lg, e0, e1, r0, r1, w0x, w1x, seg, be = outs
    return (lg, e0.reshape(T), e1.reshape(T), r0.reshape(T), r1.reshape(T),
            w0x, w1x, seg, be.reshape(NBP))


def _sc_build(e0, e1, r0, r1, seg, x):
    T = e0.shape[0]
    tpw = T // NW
    nch = tpw // LANES
    mesh = plsc.VectorSubcoreMesh(core_axis_name="c", subcore_axis_name="s",
                                  num_cores=NC, num_subcores=NS)

    @functools.partial(
        pl.kernel,
        out_type=[
            jax.ShapeDtypeStruct((NPAD, H), jnp.float32),
            jax.ShapeDtypeStruct((T,), jnp.int32),
            jax.ShapeDtypeStruct((T,), jnp.int32),
        ],
        mesh=mesh,
        scratch_types=[
            pltpu.VMEM((tpw,), jnp.int32),
            pltpu.VMEM((tpw,), jnp.int32),
            pltpu.VMEM((tpw,), jnp.int32),
            pltpu.VMEM((tpw,), jnp.int32),
            pltpu.VMEM((tpw,), jnp.int32),
            pltpu.VMEM((tpw,), jnp.int32),
            pltpu.VMEM((E, 16), jnp.int32),
            pltpu.VMEM((LANES, H), jnp.float32),
            pltpu.SemaphoreType.DMA,
        ],
    )
    def build(e0_hbm, e1_hbm, r0_hbm, r1_hbm, seg_hbm, x_hbm,
              xs_hbm, p0_hbm, p1_hbm,
              e0_v, e1_v, r0_v, r1_v, p0_v, p1_v, seg_v, xr_v, sem):
        wid = lax.axis_index("s") * NC + lax.axis_index("c")
        base = wid * tpw
        sl = pl.ds(base, tpw)
        pltpu.sync_copy(seg_hbm, seg_v)
        pltpu.sync_copy(e0_hbm.at[sl], e0_v)
        pltpu.sync_copy(e1_hbm.at[sl], e1_v)
        pltpu.sync_copy(r0_hbm.at[sl], r0_v)
        pltpu.sync_copy(r1_hbm.at[sl], r1_v)
        for c in range(nch):
            cs = pl.ds(c * LANES, LANES)
            e0c = e0_v[cs]
            e1c = e1_v[cs]
            r0c = r0_v[cs]
            r1c = r1_v[cs]
            p0c = jnp.zeros((16,), jnp.int32)
            p1c = jnp.zeros((16,), jnp.int32)
            for e in range(E):
                srow = seg_v[e, :]
                p0c = jnp.where(e0c == e, srow + r0c, p0c)
                p1c = jnp.where(e1c == e, srow + r1c, p1c)
            p0_v[cs] = p0c
            p1_v[cs] = p1c
            pltpu.sync_copy(x_hbm.at[pl.ds(base + c * LANES, LANES)], xr_v)
            pltpu.async_copy(xr_v, xs_hbm.at[p0c], sem).wait()
            pltpu.async_copy(xr_v, xs_hbm.at[p1c], sem).wait()
        pltpu.sync_copy(p0_v, p0_hbm.at[sl])
        pltpu.sync_copy(p1_v, p1_hbm.at[sl])

    return build(e0, e1, r0, r1, seg, x)


def _dgT(a, b):
    return jax.lax.dot_general(
        a, b, (((1,), (1,)), ((), ())), preferred_element_type=jnp.float32)


def _moe_body(be_ref, xs_ref, w1_ref, w3_ref, w2_ref,
              u1_ref, v1_ref, u3_ref, v3_ref, u2_ref, v2_ref, ys_ref,
              acc_ref, xv1_ref, xv3_ref):
    i = pl.program_id(0)
    b = pl.program_id(1)
    x = xs_ref[...]
    sl = pl.ds(b * BLK, BLK)

    @pl.when(i == 0)
    def _():
        xv1_ref[sl, :] = jnp.pad(
            _dgT(x, v1_ref[0]), ((0, 0), (0, 128 - R))).astype(jnp.bfloat16)
        xv3_ref[sl, :] = jnp.pad(
            _dgT(x, v3_ref[0]), ((0, 0), (0, 128 - R))).astype(jnp.bfloat16)

    xv1 = xv1_ref[sl, :][:, :R].astype(jnp.float32)
    xv3 = xv3_ref[sl, :][:, :R].astype(jnp.float32)
    gate = _dgT(x, w1_ref[0]) + _dgT(xv1, u1_ref[0])
    up = _dgT(x, w3_ref[0]) + _dgT(xv3, u3_ref[0])
    h = gate * jax.nn.sigmoid(gate) * up
    part = _dgT(h, w2_ref[0]) + _dgT(_dgT(h, v2_ref[0]), u2_ref[0])

    @pl.when(i == 0)
    def _():
        acc_ref[sl, :] = part.astype(jnp.bfloat16)

    @pl.when(i > 0)
    def _():
        acc_ref[sl, :] += part.astype(jnp.bfloat16)

    ys_ref[...] = acc_ref[sl, :].astype(jnp.float32)


def _grouped_mlp(block_expert, xs, w1, w2, w3, u1, v1, u2, v2, u3, v3):
    grid_spec = pltpu.PrefetchScalarGridSpec(
        num_scalar_prefetch=1,
        grid=(NI, NB),
        in_specs=[
            pl.BlockSpec((BLK, H), lambda i, b, be: (b, 0)),
            pl.BlockSpec((1, IB, H), lambda i, b, be: (be[b], i, 0)),
            pl.BlockSpec((1, IB, H), lambda i, b, be: (be[b], i, 0)),
            pl.BlockSpec((1, H, IB), lambda i, b, be: (be[b], 0, i)),
            pl.BlockSpec((1, IB, R), lambda i, b, be: (be[b], i, 0)),
            pl.BlockSpec((1, R, H), lambda i, b, be: (be[b], 0, 0)),
            pl.BlockSpec((1, IB, R), lambda i, b, be: (be[b], i, 0)),
            pl.BlockSpec((1, R, H), lambda i, b, be: (be[b], 0, 0)),
            pl.BlockSpec((1, H, R), lambda i, b, be: (be[b], 0, 0)),
            pl.BlockSpec((1, R, IB), lambda i, b, be: (be[b], 0, i)),
        ],
        out_specs=pl.BlockSpec((BLK, H), lambda i, b, be: (b, 0)),
        scratch_shapes=[
            pltpu.VMEM((NPAD, H), jnp.bfloat16),
            pltpu.VMEM((NPAD, 128), jnp.bfloat16),
            pltpu.VMEM((NPAD, 128), jnp.bfloat16),
        ],
    )
    return pl.pallas_call(
        _moe_body,
        grid_spec=grid_spec,
        out_shape=jax.ShapeDtypeStruct((NPAD, H), jnp.float32),
        compiler_params=pltpu.CompilerParams(
            dimension_semantics=("arbitrary", "arbitrary"),
            vmem_limit_bytes=110 * 1024 * 1024),
    )(block_expert, xs, w1, w3, w2, u1, v1, u3, v3, u2, v2)


def _sc_combine(ys, p0, p1, w0x, w1x):
    T = p0.shape[0]
    tpw = T // NW
    nch = tpw // LANES
    mesh = plsc.VectorSubcoreMesh(core_axis_name="c", subcore_axis_name="s",
                                  num_cores=NC, num_subcores=NS)

    @functools.partial(
        pl.kernel,
        out_type=jax.ShapeDtypeStruct((T, H), jnp.float32),
        mesh=mesh,
        scratch_types=[
            pltpu.VMEM((tpw,), jnp.int32),
            pltpu.VMEM((tpw,), jnp.int32),
            pltpu.VMEM((tpw, 16), jnp.float32),
            pltpu.VMEM((tpw, 16), jnp.float32),
            pltpu.VMEM((LANES, H), jnp.float32),
            pltpu.VMEM((LANES, H), jnp.float32),
            pltpu.VMEM((LANES, H), jnp.float32),
            pltpu.VMEM((LANES, H), jnp.float32),
            pltpu.VMEM((LANES, H), jnp.float32),
            pltpu.SemaphoreType.DMA,
            pltpu.SemaphoreType.DMA,
            pltpu.SemaphoreType.DMA,
            pltpu.SemaphoreType.DMA,
        ],
    )
    def combine(ys_hbm, p0_hbm, p1_hbm, w0x_hbm, w1x_hbm, out_hbm,
                p0_v, p1_v, wx0_v, wx1_v, g0a_v, g0b_v, g1a_v, g1b_v, o_v,
                s0a, s0b, s1a, s1b):
        wid = lax.axis_index("s") * NC + lax.axis_index("c")
        base = wid * tpw
        sl = pl.ds(base, tpw)
        pltpu.sync_copy(p0_hbm.at[sl], p0_v)
        pltpu.sync_copy(p1_hbm.at[sl], p1_v)
        pltpu.sync_copy(w0x_hbm.at[sl], wx0_v)
        pltpu.sync_copy(w1x_hbm.at[sl], wx1_v)
        g0 = (g0a_v, g0b_v)
        g1 = (g1a_v, g1b_v)
        s0 = (s0a, s0b)
        s1 = (s1a, s1b)

        def issue(c):
            slot = c % 2
            cs = pl.ds(c * LANES, LANES)
            cp0 = pltpu.async_copy(ys_hbm.at[p0_v[cs]], g0[slot], s0[slot])
            cp1 = pltpu.async_copy(ys_hbm.at[p1_v[cs]], g1[slot], s1[slot])
            return cp0, cp1

        inflight = {0: issue(0)}
        for c in range(nch):
            slot = c % 2
            cp0, cp1 = inflight.pop(c)
            cp0.wait()
            cp1.wait()
            if c + 1 < nch:
                inflight[c + 1] = issue(c + 1)
            for r in range(LANES):
                w0s = wx0_v[c * LANES + r, :]
                w1s = wx1_v[c * LANES + r, :]

                def body(j, _):
                    js = pl.ds(j * LANES, LANES)
                    o_v[r, js] = (g0[slot][r, js] * w0s
                                  + g1[slot][r, js] * w1s)
                    return 0

                lax.fori_loop(0, H // LANES, body, 0)
            pltpu.sync_copy(o_v, out_hbm.at[pl.ds(base + c * LANES, LANES)])

    return combine(ys, p0, p1, w0x, w1x)


def kernel(hidden_states, gate_w, w1, w2, w3, u1, v1, u2, v2, u3, v3):
    b, s, hd = hidden_states.shape
    x = hidden_states.reshape(-1, hd)

    logits, e0, e1, r0, r1, w0x, w1x, seg, be_tab = _route(x, gate_w)
    xs, p0, p1 = _sc_build(e0, e1, r0, r1, seg, x)
    ys = _grouped_mlp(be_tab[:NB], xs, w1, w2, w3, u1, v1, u2, v2, u3, v3)
    final = _sc_combine(ys, p0, p1, w0x, w1x)
    return final.reshape(b, s, hd), logits
